# baseline (speedup 1.0000x reference)
import functools

import jax
import jax.numpy as jnp
from jax.experimental import pallas as pl
from jax.experimental.pallas import tpu as pltpu

F32 = jnp.float32
BF16 = jnp.bfloat16


def _sig(x):
    return 0.5 * jnp.tanh(0.5 * x) + 0.5


def _main_body(T, Nb, C, H, Hh,
               xt_ref, y_ref,
               w1fih, w1fhh, b1f, w1bih, w1bhh, b1b,
               w2fih, w2fhh, b2f, w2bih, w2bhh, b2b,
               lin1f, lin1b, lin1bias, lin2f, lin2b, lin2bias,
               outw, outb, bboxw, bboxb,
               scores_ref, deltas_ref,
               acc_ref, uf_ref, ub_ref, hsf_ref, hsb_ref, z2_ref):
    G = 4 * H
    h = pl.program_id(1)

    @pl.when(h == 0)
    def _():
        acc_ref[...] = xt_ref[0]

    @pl.when(h > 0)
    def _():
        acc_ref[...] += xt_ref[0]

    def bilstm(src, wfih, wfhh, bf, wbih, wbhh, bb):
        uf = jnp.dot(src, wfih[...], preferred_element_type=F32) + bf[...]
        uf_ref[...] = uf.reshape(T, Nb, G)
        ub = jnp.dot(src, wbih[...], preferred_element_type=F32) + bb[...]
        ub_ref[...] = ub.reshape(T, Nb, G)
        whf = wfhh[...]
        whb = wbhh[...]
        hf = jnp.zeros((Nb, H), BF16)
        cf = jnp.zeros((Nb, H), F32)
        hb = jnp.zeros((Nb, H), BF16)
        cb = jnp.zeros((Nb, H), F32)
        for k in range(T):
            tb = T - 1 - k
            gf = uf_ref[k] + jnp.dot(hf, whf, preferred_element_type=F32)
            gb = ub_ref[tb] + jnp.dot(hb, whb, preferred_element_type=F32)
            cf = (_sig(gf[:, 1 * H:2 * H]) * cf
                  + _sig(gf[:, 0 * H:1 * H]) * jnp.tanh(gf[:, 2 * H:3 * H]))
            cb = (_sig(gb[:, 1 * H:2 * H]) * cb
                  + _sig(gb[:, 0 * H:1 * H]) * jnp.tanh(gb[:, 2 * H:3 * H]))
            hf = (_sig(gf[:, 3 * H:4 * H]) * jnp.tanh(cf)).astype(BF16)
            hb = (_sig(gb[:, 3 * H:4 * H]) * jnp.tanh(cb)).astype(BF16)
            hsf_ref[k] = hf
            hsb_ref[tb] = hb

    def lin(wf, wb, bias):
        return (jnp.dot(hsf_ref[...].reshape(T * Nb, H), wf[...],
                        preferred_element_type=F32)
                + jnp.dot(hsb_ref[...].reshape(T * Nb, H), wb[...],
                          preferred_element_type=F32)
                + bias[...])

    @pl.when(h == Hh - 1)
    def _():
        z1 = (acc_ref[...] * (1.0 / Hh)).astype(BF16).reshape(T * Nb, C)
        bilstm(z1, w1fih, w1fhh, b1f, w1bih, w1bhh, b1b)
        rec1 = lin(lin1f, lin1b, lin1bias)
        z2_ref[...] = rec1.astype(BF16).reshape(T, Nb, H)

        bilstm(z2_ref[...].reshape(T * Nb, H),
               w2fih, w2fhh, b2f, w2bih, w2bhh, b2b)
        rec2 = lin(lin2f, lin2b, lin2bias)

        sc = (jnp.dot(rec2.astype(BF16), outw[...],
                      preferred_element_type=F32) + outb[...])
        sc3 = sc.reshape(T, Nb, sc.shape[-1])
        for t in range(T):
            scores_ref[:, t, :] = sc3[t]

        deltas_ref[...] = (jnp.dot(y_ref[...].astype(BF16), bboxw[...],
                                   preferred_element_type=F32) + bboxb[...])


def kernel(x, y,
           l1_f_Wih, l1_f_Whh, l1_f_bih, l1_f_bhh,
           l1_b_Wih, l1_b_Whh, l1_b_bih, l1_b_bhh,
           l1_lin_W, l1_lin_b,
           l2_f_Wih, l2_f_Whh, l2_f_bih, l2_f_bhh,
           l2_b_Wih, l2_b_Whh, l2_b_bih, l2_b_bhh,
           l2_lin_W, l2_lin_b,
           out_W, out_b, bbox_W, bbox_b):
    N, C, Hh, Ww = x.shape
    T = Ww
    H = l1_f_Whh.shape[1]
    G = 4 * H
    NC = out_W.shape[0]
    FC = y.shape[1]
    Nb = 200
    grid = (N // Nb, Hh)

    xt = jnp.transpose(x, (2, 3, 0, 1))

    def prep(wih, whh, bih, bhh):
        return (wih.T.astype(BF16), whh.T.astype(BF16),
                (bih + bhh).reshape(1, G))

    w1fih, w1fhh, b1f = prep(l1_f_Wih, l1_f_Whh, l1_f_bih, l1_f_bhh)
    w1bih, w1bhh, b1b = prep(l1_b_Wih, l1_b_Whh, l1_b_bih, l1_b_bhh)
    w2fih, w2fhh, b2f = prep(l2_f_Wih, l2_f_Whh, l2_f_bih, l2_f_bhh)
    w2bih, w2bhh, b2b = prep(l2_b_Wih, l2_b_Whh, l2_b_bih, l2_b_bhh)
    lin1f = l1_lin_W[:, :H].T.astype(BF16)
    lin1b = l1_lin_W[:, H:].T.astype(BF16)
    lin2f = l2_lin_W[:, :H].T.astype(BF16)
    lin2b = l2_lin_W[:, H:].T.astype(BF16)
    lin1bias = l1_lin_b.reshape(1, H)
    lin2bias = l2_lin_b.reshape(1, H)
    outw = out_W.T.astype(BF16)
    outb = out_b.reshape(1, NC)
    bboxw = bbox_W.T.astype(BF16)
    bboxb = bbox_b.reshape(1, 4)

    full = lambda shape: pl.BlockSpec(shape, lambda i, h: tuple(0 for _ in shape))
    in_specs = [
        pl.BlockSpec((1, Ww, Nb, C), lambda i, h: (h, 0, i, 0)),
        pl.BlockSpec((Nb, FC), lambda i, h: (i, 0)),
        full((C, G)), full((H, G)), full((1, G)),
        full((C, G)), full((H, G)), full((1, G)),
        full((H, G)), full((H, G)), full((1, G)),
        full((H, G)), full((H, G)), full((1, G)),
        full((H, H)), full((H, H)), full((1, H)),
        full((H, H)), full((H, H)), full((1, H)),
        full((H, NC)), full((1, NC)),
        full((FC, 4)), full((1, 4)),
    ]
    out_specs = [
        pl.BlockSpec((Nb, T, NC), lambda i, h: (i, 0, 0)),
        pl.BlockSpec((Nb, 4), lambda i, h: (i, 0)),
    ]
    out_shape = [
        jax.ShapeDtypeStruct((N, T, NC), F32),
        jax.ShapeDtypeStruct((N, 4), F32),
    ]
    scratch_shapes = [
        pltpu.VMEM((Ww, Nb, C), F32),
        pltpu.VMEM((T, Nb, G), F32),
        pltpu.VMEM((T, Nb, G), F32),
        pltpu.VMEM((T, Nb, H), BF16),
        pltpu.VMEM((T, Nb, H), BF16),
        pltpu.VMEM((T, Nb, H), BF16),
    ]

    scores, deltas = pl.pallas_call(
        functools.partial(_main_body, T, Nb, C, H, Hh),
        grid=grid,
        in_specs=in_specs,
        out_specs=out_specs,
        out_shape=out_shape,
        scratch_shapes=scratch_shapes,
    )(xt, y,
      w1fih, w1fhh, b1f, w1bih, w1bhh, b1b,
      w2fih, w2fhh, b2f, w2bih, w2bhh, b2b,
      lin1f, lin1b, lin1bias, lin2f, lin2b, lin2bias,
      outw, outb, bboxw, bboxb)

    return (scores, deltas)


# speedup vs baseline: 1.0975x; 1.0975x over previous
import functools

import jax
import jax.numpy as jnp
from jax import lax
from jax.experimental import pallas as pl
from jax.experimental.pallas import tpu as pltpu

F32 = jnp.float32
BF16 = jnp.bfloat16

_DNT = (((1,), (1,)), ((), ()))


def _pool_body(inv_h, x_ref, o_ref):
    s = jnp.sum(x_ref[...], axis=0)
    o_ref[...] = (s * inv_h).astype(BF16)


def _avg_pool(x, Hh, Ww):
    N, C = x.shape[0], x.shape[1]
    xt = jnp.transpose(x, (2, 3, 0, 1))
    nb = 40
    grid = N // nb
    return pl.pallas_call(
        functools.partial(_pool_body, 1.0 / float(Hh)),
        grid=(grid,),
        in_specs=[
            pl.BlockSpec((Hh, Ww, nb, C), lambda i: (0, 0, i, 0)),
        ],
        out_specs=pl.BlockSpec((Ww, nb, C), lambda i: (0, i, 0)),
        out_shape=jax.ShapeDtypeStruct((Ww, N, C), BF16),
    )(xt)


def _sig(x):
    return 0.5 * jnp.tanh(0.5 * x) + 0.5


def _main_body(T, Nb, C, H,
               zt_ref, y_ref,
               w1fih, w1fhh, b1f, w1bih, w1bhh, b1b,
               w2fih, w2fhh, b2f, w2bih, w2bhh, b2b,
               lin1, lin1bias, lin2, lin2bias,
               outw, outb, bboxw, bboxb,
               scores_ref, deltas_ref,
               uf_ref, ub_ref, hsf_ref, hsb_ref, z2_ref):
    G = 4 * H

    def bilstm(src, wfih, wfhh, bf, wbih, wbhh, bb):
        uf = lax.dot_general(src, wfih[...], _DNT,
                             preferred_element_type=F32) + bf[...]
        uf_ref[...] = uf.reshape(T, Nb, G)
        ub = lax.dot_general(src, wbih[...], _DNT,
                             preferred_element_type=F32) + bb[...]
        ub_ref[...] = ub.reshape(T, Nb, G)
        whf = wfhh[...]
        whb = wbhh[...]
        hf = jnp.zeros((Nb, H), BF16)
        cf = jnp.zeros((Nb, H), F32)
        hb = jnp.zeros((Nb, H), BF16)
        cb = jnp.zeros((Nb, H), F32)
        for k in range(T):
            tb = T - 1 - k
            gf = uf_ref[k] + lax.dot_general(hf, whf, _DNT,
                                             preferred_element_type=F32)
            gb = ub_ref[tb] + lax.dot_general(hb, whb, _DNT,
                                              preferred_element_type=F32)
            cf = (_sig(gf[:, 1 * H:2 * H]) * cf
                  + _sig(gf[:, 0 * H:1 * H]) * jnp.tanh(gf[:, 2 * H:3 * H]))
            cb = (_sig(gb[:, 1 * H:2 * H]) * cb
                  + _sig(gb[:, 0 * H:1 * H]) * jnp.tanh(gb[:, 2 * H:3 * H]))
            hf = (_sig(gf[:, 3 * H:4 * H]) * jnp.tanh(cf)).astype(BF16)
            hb = (_sig(gb[:, 3 * H:4 * H]) * jnp.tanh(cb)).astype(BF16)
            hsf_ref[k] = hf
            hsb_ref[tb] = hb

    def lin(linw, bias):
        wf = linw[...][:, :H]
        wb = linw[...][:, H:]
        return (lax.dot_general(hsf_ref[...].reshape(T * Nb, H), wf, _DNT,
                                preferred_element_type=F32)
                + lax.dot_general(hsb_ref[...].reshape(T * Nb, H), wb, _DNT,
                                  preferred_element_type=F32)
                + bias[...])

    z1 = zt_ref[...].reshape(T * Nb, C)
    bilstm(z1, w1fih, w1fhh, b1f, w1bih, w1bhh, b1b)
    rec1 = lin(lin1, lin1bias)
    z2_ref[...] = rec1.astype(BF16).reshape(T, Nb, H)

    bilstm(z2_ref[...].reshape(T * Nb, H), w2fih, w2fhh, b2f, w2bih, w2bhh, b2b)
    rec2 = lin(lin2, lin2bias)

    sc = (lax.dot_general(rec2.astype(BF16), outw[...], _DNT,
                          preferred_element_type=F32) + outb[...])
    sc3 = sc.reshape(T, Nb, sc.shape[-1])
    for t in range(T):
        scores_ref[:, t, :] = sc3[t]

    deltas_ref[...] = (lax.dot_general(y_ref[...].astype(BF16), bboxw[...],
                                       _DNT,
                                       preferred_element_type=F32) + bboxb[...])


def kernel(x, y,
           l1_f_Wih, l1_f_Whh, l1_f_bih, l1_f_bhh,
           l1_b_Wih, l1_b_Whh, l1_b_bih, l1_b_bhh,
           l1_lin_W, l1_lin_b,
           l2_f_Wih, l2_f_Whh, l2_f_bih, l2_f_bhh,
           l2_b_Wih, l2_b_Whh, l2_b_bih, l2_b_bhh,
           l2_lin_W, l2_lin_b,
           out_W, out_b, bbox_W, bbox_b):
    N, C, Hh, Ww = x.shape
    T = Ww
    H = l1_f_Whh.shape[1]
    G = 4 * H
    NC = out_W.shape[0]
    FC = y.shape[1]
    Nb = 200
    grid = N // Nb

    zt = _avg_pool(x, Hh, Ww)

    def prep(wih, whh, bih, bhh):
        return (wih.astype(BF16), whh.astype(BF16), (bih + bhh).reshape(1, G))

    w1fih, w1fhh, b1f = prep(l1_f_Wih, l1_f_Whh, l1_f_bih, l1_f_bhh)
    w1bih, w1bhh, b1b = prep(l1_b_Wih, l1_b_Whh, l1_b_bih, l1_b_bhh)
    w2fih, w2fhh, b2f = prep(l2_f_Wih, l2_f_Whh, l2_f_bih, l2_f_bhh)
    w2bih, w2bhh, b2b = prep(l2_b_Wih, l2_b_Whh, l2_b_bih, l2_b_bhh)
    lin1 = l1_lin_W.astype(BF16)
    lin2 = l2_lin_W.astype(BF16)
    lin1bias = l1_lin_b.reshape(1, H)
    lin2bias = l2_lin_b.reshape(1, H)
    outw = out_W.astype(BF16)
    outb = out_b.reshape(1, NC)
    bboxw = bbox_W.astype(BF16)
    bboxb = bbox_b.reshape(1, 4)

    full = lambda shape: pl.BlockSpec(shape, lambda i: tuple(0 for _ in shape))
    in_specs = [
        pl.BlockSpec((T, Nb, C), lambda i: (0, i, 0)),
        pl.BlockSpec((Nb, FC), lambda i: (i, 0)),
        full((G, C)), full((G, H)), full((1, G)),
        full((G, C)), full((G, H)), full((1, G)),
        full((G, H)), full((G, H)), full((1, G)),
        full((G, H)), full((G, H)), full((1, G)),
        full((H, 2 * H)), full((1, H)),
        full((H, 2 * H)), full((1, H)),
        full((NC, H)), full((1, NC)),
        full((4, FC)), full((1, 4)),
    ]
    out_specs = [
        pl.BlockSpec((Nb, T, NC), lambda i: (i, 0, 0)),
        pl.BlockSpec((Nb, 4), lambda i: (i, 0)),
    ]
    out_shape = [
        jax.ShapeDtypeStruct((N, T, NC), F32),
        jax.ShapeDtypeStruct((N, 4), F32),
    ]
    scratch_shapes = [
        pltpu.VMEM((T, Nb, G), F32),
        pltpu.VMEM((T, Nb, G), F32),
        pltpu.VMEM((T, Nb, H), BF16),
        pltpu.VMEM((T, Nb, H), BF16),
        pltpu.VMEM((T, Nb, H), BF16),
    ]

    scores, deltas = pl.pallas_call(
        functools.partial(_main_body, T, Nb, C, H),
        grid=(grid,),
        in_specs=in_specs,
        out_specs=out_specs,
        out_shape=out_shape,
        scratch_shapes=scratch_shapes,
    )(zt, y,
      w1fih, w1fhh, b1f, w1bih, w1bhh, b1b,
      w2fih, w2fhh, b2f, w2bih, w2bhh, b2b,
      lin1, lin1bias, lin2, lin2bias,
      outw, outb, bboxw, bboxb)

    return (scores, deltas)


# speedup vs baseline: 1.2383x; 1.1283x over previous
import functools

import jax
import jax.numpy as jnp
from jax import lax
from jax.experimental import pallas as pl
from jax.experimental.pallas import tpu as pltpu

F32 = jnp.float32
BF16 = jnp.bfloat16

_DNT = (((1,), (1,)), ((), ()))


def _pool_body(inv_h, x_ref, o_ref):
    s = jnp.sum(x_ref[...], axis=0)
    o_ref[...] = (s * inv_h).astype(BF16)


def _avg_pool(x, Hh, Ww):
    N, C = x.shape[0], x.shape[1]
    xt = jnp.transpose(x, (2, 3, 0, 1))
    nb = 40
    grid = N // nb
    return pl.pallas_call(
        functools.partial(_pool_body, 1.0 / float(Hh)),
        grid=(grid,),
        in_specs=[
            pl.BlockSpec((Hh, Ww, nb, C), lambda i: (0, 0, i, 0)),
        ],
        out_specs=pl.BlockSpec((Ww, nb, C), lambda i: (0, i, 0)),
        out_shape=jax.ShapeDtypeStruct((Ww, N, C), BF16),
    )(xt)


def _sig(x):
    return 0.5 * jnp.tanh(0.5 * x) + 0.5


def _main_body(T, Nb, C, H,
               zt_ref, y_ref,
               w1fih, w1fhh, b1fi, b1fh, w1bih, w1bhh, b1bi, b1bh,
               w2fih, w2fhh, b2fi, b2fh, w2bih, w2bhh, b2bi, b2bh,
               lin1, lin1bias, lin2, lin2bias,
               outw, outb, bboxw, bboxb,
               scores_ref, deltas_ref,
               uf_ref, ub_ref, hsf_ref, hsb_ref, z2_ref,
               w1fih_s, w1fhh_s, w1bih_s, w1bhh_s,
               w2fih_s, w2fhh_s, w2bih_s, w2bhh_s,
               lin1_s, lin2_s, outw_s, bboxw_s,
               b1f_s, b1b_s, b2f_s, b2b_s):
    G = 4 * H

    @pl.when(pl.program_id(0) == 0)
    def _():
        w1fih_s[...] = w1fih[...].astype(BF16)
        w1fhh_s[...] = w1fhh[...].astype(BF16)
        w1bih_s[...] = w1bih[...].astype(BF16)
        w1bhh_s[...] = w1bhh[...].astype(BF16)
        w2fih_s[...] = w2fih[...].astype(BF16)
        w2fhh_s[...] = w2fhh[...].astype(BF16)
        w2bih_s[...] = w2bih[...].astype(BF16)
        w2bhh_s[...] = w2bhh[...].astype(BF16)
        lin1_s[...] = lin1[...].astype(BF16)
        lin2_s[...] = lin2[...].astype(BF16)
        outw_s[...] = outw[...].astype(BF16)
        bboxw_s[...] = bboxw[...].astype(BF16)
        b1f_s[...] = b1fi[...] + b1fh[...]
        b1b_s[...] = b1bi[...] + b1bh[...]
        b2f_s[...] = b2fi[...] + b2fh[...]
        b2b_s[...] = b2bi[...] + b2bh[...]

    def bilstm(src, wfih_s, wfhh_s, bf_s, wbih_s, wbhh_s, bb_s):
        uf = lax.dot_general(src, wfih_s[...], _DNT,
                             preferred_element_type=F32) + bf_s[...]
        uf_ref[...] = uf.astype(BF16).reshape(T, Nb, G)
        ub = lax.dot_general(src, wbih_s[...], _DNT,
                             preferred_element_type=F32) + bb_s[...]
        ub_ref[...] = ub.astype(BF16).reshape(T, Nb, G)
        whf = wfhh_s[...]
        whb = wbhh_s[...]
        hf = jnp.zeros((Nb, H), BF16)
        cf = jnp.zeros((Nb, H), F32)
        hb = jnp.zeros((Nb, H), BF16)
        cb = jnp.zeros((Nb, H), F32)
        for k in range(T):
            tb = T - 1 - k
            gf = uf_ref[k] + lax.dot_general(hf, whf, _DNT,
                                             preferred_element_type=F32)
            gb = ub_ref[tb] + lax.dot_general(hb, whb, _DNT,
                                              preferred_element_type=F32)
            cf = (_sig(gf[:, 1 * H:2 * H]) * cf
                  + _sig(gf[:, 0 * H:1 * H]) * jnp.tanh(gf[:, 2 * H:3 * H]))
            cb = (_sig(gb[:, 1 * H:2 * H]) * cb
                  + _sig(gb[:, 0 * H:1 * H]) * jnp.tanh(gb[:, 2 * H:3 * H]))
            hf = (_sig(gf[:, 3 * H:4 * H]) * jnp.tanh(cf)).astype(BF16)
            hb = (_sig(gb[:, 3 * H:4 * H]) * jnp.tanh(cb)).astype(BF16)
            hsf_ref[k] = hf
            hsb_ref[tb] = hb

    def lin(lin_s, bias):
        wf = lin_s[...][:, :H]
        wb = lin_s[...][:, H:]
        return (lax.dot_general(hsf_ref[...].reshape(T * Nb, H), wf, _DNT,
                                preferred_element_type=F32)
                + lax.dot_general(hsb_ref[...].reshape(T * Nb, H), wb, _DNT,
                                  preferred_element_type=F32)
                + bias[...])

    z1 = zt_ref[...].reshape(T * Nb, C)
    bilstm(z1, w1fih_s, w1fhh_s, b1f_s, w1bih_s, w1bhh_s, b1b_s)
    rec1 = lin(lin1_s, lin1bias)
    z2_ref[...] = rec1.astype(BF16).reshape(T, Nb, H)

    bilstm(z2_ref[...].reshape(T * Nb, H),
           w2fih_s, w2fhh_s, b2f_s, w2bih_s, w2bhh_s, b2b_s)
    rec2 = lin(lin2_s, lin2bias)

    sc = (lax.dot_general(rec2.astype(BF16), outw_s[...], _DNT,
                          preferred_element_type=F32) + outb[...])
    sc3 = sc.reshape(T, Nb, sc.shape[-1])
    for t in range(T):
        scores_ref[:, t, :] = sc3[t]

    deltas_ref[...] = (lax.dot_general(y_ref[...].astype(BF16), bboxw_s[...],
                                       _DNT,
                                       preferred_element_type=F32) + bboxb[...])


def kernel(x, y,
           l1_f_Wih, l1_f_Whh, l1_f_bih, l1_f_bhh,
           l1_b_Wih, l1_b_Whh, l1_b_bih, l1_b_bhh,
           l1_lin_W, l1_lin_b,
           l2_f_Wih, l2_f_Whh, l2_f_bih, l2_f_bhh,
           l2_b_Wih, l2_b_Whh, l2_b_bih, l2_b_bhh,
           l2_lin_W, l2_lin_b,
           out_W, out_b, bbox_W, bbox_b):
    N, C, Hh, Ww = x.shape
    T = Ww
    H = l1_f_Whh.shape[1]
    G = 4 * H
    NC = out_W.shape[0]
    FC = y.shape[1]
    Nb = 200
    grid = N // Nb

    zt = _avg_pool(x, Hh, Ww)

    b1fi = l1_f_bih.reshape(1, G)
    b1fh = l1_f_bhh.reshape(1, G)
    b1bi = l1_b_bih.reshape(1, G)
    b1bh = l1_b_bhh.reshape(1, G)
    b2fi = l2_f_bih.reshape(1, G)
    b2fh = l2_f_bhh.reshape(1, G)
    b2bi = l2_b_bih.reshape(1, G)
    b2bh = l2_b_bhh.reshape(1, G)
    lin1bias = l1_lin_b.reshape(1, H)
    lin2bias = l2_lin_b.reshape(1, H)
    outb = out_b.reshape(1, NC)
    bboxb = bbox_b.reshape(1, 4)

    full = lambda shape: pl.BlockSpec(shape, lambda i: tuple(0 for _ in shape))
    in_specs = [
        pl.BlockSpec((T, Nb, C), lambda i: (0, i, 0)),
        pl.BlockSpec((Nb, FC), lambda i: (i, 0)),
        full((G, C)), full((G, H)), full((1, G)), full((1, G)),
        full((G, C)), full((G, H)), full((1, G)), full((1, G)),
        full((G, H)), full((G, H)), full((1, G)), full((1, G)),
        full((G, H)), full((G, H)), full((1, G)), full((1, G)),
        full((H, 2 * H)), full((1, H)),
        full((H, 2 * H)), full((1, H)),
        full((NC, H)), full((1, NC)),
        full((4, FC)), full((1, 4)),
    ]
    out_specs = [
        pl.BlockSpec((Nb, T, NC), lambda i: (i, 0, 0)),
        pl.BlockSpec((Nb, 4), lambda i: (i, 0)),
    ]
    out_shape = [
        jax.ShapeDtypeStruct((N, T, NC), F32),
        jax.ShapeDtypeStruct((N, 4), F32),
    ]
    scratch_shapes = [
        pltpu.VMEM((T, Nb, G), BF16),
        pltpu.VMEM((T, Nb, G), BF16),
        pltpu.VMEM((T, Nb, H), BF16),
        pltpu.VMEM((T, Nb, H), BF16),
        pltpu.VMEM((T, Nb, H), BF16),
        pltpu.VMEM((G, C), BF16), pltpu.VMEM((G, H), BF16),
        pltpu.VMEM((G, C), BF16), pltpu.VMEM((G, H), BF16),
        pltpu.VMEM((G, H), BF16), pltpu.VMEM((G, H), BF16),
        pltpu.VMEM((G, H), BF16), pltpu.VMEM((G, H), BF16),
        pltpu.VMEM((H, 2 * H), BF16), pltpu.VMEM((H, 2 * H), BF16),
        pltpu.VMEM((NC, H), BF16), pltpu.VMEM((4, FC), BF16),
        pltpu.VMEM((1, G), F32), pltpu.VMEM((1, G), F32),
        pltpu.VMEM((1, G), F32), pltpu.VMEM((1, G), F32),
    ]

    scores, deltas = pl.pallas_call(
        functools.partial(_main_body, T, Nb, C, H),
        grid=(grid,),
        in_specs=in_specs,
        out_specs=out_specs,
        out_shape=out_shape,
        scratch_shapes=scratch_shapes,
    )(zt, y,
      l1_f_Wih, l1_f_Whh, b1fi, b1fh, l1_b_Wih, l1_b_Whh, b1bi, b1bh,
      l2_f_Wih, l2_f_Whh, b2fi, b2fh, l2_b_Wih, l2_b_Whh, b2bi, b2bh,
      l1_lin_W, lin1bias, l2_lin_W, lin2bias,
      out_W, outb, bbox_W, bboxb)

    return (scores, deltas)
